# baseline (speedup 1.0000x reference)
import functools

import jax
import jax.numpy as jnp
from jax import lax
from jax.experimental import pallas as pl
from jax.experimental.pallas import tpu as pltpu
from jax.experimental.pallas import tpu_sc as plsc

BATCH = 16384
SEQ = 20
EMBED_DIM = 32
HALF = 16
LANES = 128

NUM_CORES = 2
NUM_SUBCORES = 16
NUM_WORKERS = NUM_CORES * NUM_SUBCORES
B_PER_W = BATCH // NUM_WORKERS
IDX_PER_W = B_PER_W * SEQ

CHUNK_B = 64
CHUNK_IDX = CHUNK_B * SEQ
CHUNK_STREAMS = CHUNK_IDX // LANES
NUM_CHUNKS = B_PER_W // CHUNK_B

_MESH = plsc.VectorSubcoreMesh(core_axis_name="c", subcore_axis_name="s")


def _worker_id():
    return lax.axis_index("s") * NUM_CORES + lax.axis_index("c")


@functools.partial(
    pl.kernel,
    out_type=jax.ShapeDtypeStruct((BATCH * SEQ,), jnp.int32),
    mesh=_MESH,
    compiler_params=pltpu.CompilerParams(use_tc_tiling_on_sc=True),
    scratch_types=[
        pltpu.VMEM((B_PER_W, SEQ), jnp.int32),
        pltpu.VMEM((IDX_PER_W,), jnp.int32),
    ],
)
def _sc_flatten(x_hbm, out_hbm, xin_v, xout_v):
    wid = _worker_id()
    pltpu.sync_copy(x_hbm.at[pl.ds(wid * B_PER_W, B_PER_W), :], xin_v)

    @pl.loop(0, B_PER_W)
    def _(r):
        a = xin_v[r, pl.ds(0, HALF)]
        b = xin_v[r, pl.ds(SEQ - HALF, HALF)]
        xout_v[pl.ds(r * SEQ, HALF)] = a
        xout_v[pl.ds(r * SEQ + (SEQ - HALF), HALF)] = b

    pltpu.sync_copy(xout_v, out_hbm.at[pl.ds(wid * IDX_PER_W, IDX_PER_W)])


@functools.partial(
    pl.kernel,
    out_type=jax.ShapeDtypeStruct((BATCH, EMBED_DIM), jnp.float32),
    mesh=_MESH,
    compiler_params=pltpu.CompilerParams(use_tc_tiling_on_sc=False),
    scratch_types=[
        pltpu.VMEM((IDX_PER_W,), jnp.int32),
        pltpu.VMEM((CHUNK_IDX, EMBED_DIM), jnp.float32),
        pltpu.VMEM((CHUNK_IDX, EMBED_DIM), jnp.float32),
        pltpu.VMEM((B_PER_W, EMBED_DIM), jnp.float32),
        pltpu.SemaphoreType.DMA,
        pltpu.SemaphoreType.DMA,
    ],
)
def _sc_embed_sum(table_hbm, idx_hbm, out_hbm, idx_v, rows0, rows1, out_v,
                  sem0, sem1):
    wid = _worker_id()
    base_b = wid * B_PER_W
    pltpu.sync_copy(idx_hbm.at[pl.ds(wid * IDX_PER_W, IDX_PER_W)], idx_v)

    rows = (rows0, rows1)
    sems = (sem0, sem1)

    def fire(c):
        buf, sem = rows[c % 2], sems[c % 2]
        cps = []
        for g in range(CHUNK_STREAMS):
            cps.append(pltpu.async_copy(
                table_hbm.at[idx_v.at[pl.ds(c * CHUNK_IDX + g * LANES, LANES)]],
                buf.at[pl.ds(g * LANES, LANES)],
                sem))
        return cps

    pending = fire(0)
    for c in range(NUM_CHUNKS):
        for cp in pending:
            cp.wait()
        if c + 1 < NUM_CHUNKS:
            pending = fire(c + 1)
        buf = rows[c % 2]

        @pl.loop(0, CHUNK_B)
        def _(b, _c=c, _buf=buf):
            r0 = b * SEQ
            acc0 = _buf[r0, pl.ds(0, HALF)]
            acc1 = _buf[r0, pl.ds(HALF, HALF)]
            for l in range(1, SEQ):
                acc0 = acc0 + _buf[r0 + l, pl.ds(0, HALF)]
                acc1 = acc1 + _buf[r0 + l, pl.ds(HALF, HALF)]
            ob = _c * CHUNK_B + b
            out_v[ob, pl.ds(0, HALF)] = acc0
            out_v[ob, pl.ds(HALF, HALF)] = acc1

    pltpu.sync_copy(out_v, out_hbm.at[pl.ds(base_b, B_PER_W)])


TABLE_ROWS = 100000
DETILE_WORKERS = 25
ROWS_PER_DW = TABLE_ROWS // DETILE_WORKERS
DET_CHUNK = 200
DET_CHUNKS = ROWS_PER_DW // DET_CHUNK
DET_WORDS = DET_CHUNK * EMBED_DIM


@functools.partial(
    pl.kernel,
    out_type=jax.ShapeDtypeStruct((TABLE_ROWS * EMBED_DIM,), jnp.float32),
    mesh=_MESH,
    compiler_params=pltpu.CompilerParams(use_tc_tiling_on_sc=True),
    scratch_types=[
        pltpu.VMEM((DET_CHUNK, EMBED_DIM), jnp.float32),
        pltpu.VMEM((DET_CHUNK, EMBED_DIM), jnp.float32),
        pltpu.VMEM((DET_WORDS,), jnp.float32),
        pltpu.VMEM((DET_WORDS,), jnp.float32),
        pltpu.SemaphoreType.DMA,
        pltpu.SemaphoreType.DMA,
    ],
)
def _sc_detile_table(t_hbm, out_hbm, tv0, tv1, ov0, ov1, sem0, sem1):
    wid = _worker_id()

    @pl.when(wid < DETILE_WORKERS)
    def _():
        base_r = wid * ROWS_PER_DW
        tvs, ovs, sems = (tv0, tv1), (ov0, ov1), (sem0, sem1)

        def fire(c):
            return pltpu.async_copy(
                t_hbm.at[pl.ds(base_r + c * DET_CHUNK, DET_CHUNK), :],
                tvs[c % 2], sems[c % 2])

        pending = fire(0)
        for c in range(DET_CHUNKS):
            pending.wait()
            if c + 1 < DET_CHUNKS:
                pending = fire(c + 1)
            tv, ov = tvs[c % 2], ovs[c % 2]

            @pl.loop(0, DET_CHUNK)
            def _(r, _tv=tv, _ov=ov):
                _ov[pl.ds(r * EMBED_DIM, HALF)] = _tv[r, pl.ds(0, HALF)]
                _ov[pl.ds(r * EMBED_DIM + HALF, HALF)] = _tv[r, pl.ds(HALF, HALF)]

            pltpu.sync_copy(
                ov, out_hbm.at[pl.ds((base_r + c * DET_CHUNK) * EMBED_DIM,
                                     DET_WORDS)])


def kernel(x, action_emb):
    x_flat = _sc_flatten(x.astype(jnp.int32))
    table_lin = _sc_detile_table(action_emb)
    return _sc_embed_sum(table_lin.reshape(TABLE_ROWS, EMBED_DIM), x_flat)


# speedup vs baseline: 1.2828x; 1.2828x over previous
import functools

import jax
import jax.numpy as jnp
from jax import lax
from jax.experimental import pallas as pl
from jax.experimental.pallas import tpu as pltpu
from jax.experimental.pallas import tpu_sc as plsc

BATCH = 16384
SEQ = 20
EMBED_DIM = 32
HALF = 16
LANES = 128

NUM_CORES = 2
NUM_SUBCORES = 16
NUM_WORKERS = NUM_CORES * NUM_SUBCORES
B_PER_W = BATCH // NUM_WORKERS
IDX_PER_W = B_PER_W * SEQ

CHUNK_B = 64
CHUNK_IDX = CHUNK_B * SEQ
CHUNK_STREAMS = CHUNK_IDX // LANES
NUM_CHUNKS = B_PER_W // CHUNK_B

_MESH = plsc.VectorSubcoreMesh(core_axis_name="c", subcore_axis_name="s")


def _worker_id():
    return lax.axis_index("s") * NUM_CORES + lax.axis_index("c")


@functools.partial(
    pl.kernel,
    out_type=jax.ShapeDtypeStruct((BATCH * SEQ,), jnp.int32),
    mesh=_MESH,
    compiler_params=pltpu.CompilerParams(use_tc_tiling_on_sc=True),
    scratch_types=[
        pltpu.VMEM((B_PER_W, SEQ), jnp.int32),
        pltpu.VMEM((IDX_PER_W,), jnp.int32),
    ],
)
def _sc_flatten(x_hbm, out_hbm, xin_v, xout_v):
    wid = _worker_id()
    pltpu.sync_copy(x_hbm.at[pl.ds(wid * B_PER_W, B_PER_W), :], xin_v)

    @pl.loop(0, B_PER_W)
    def _(r):
        a = xin_v[r, pl.ds(0, HALF)]
        b = xin_v[r, pl.ds(SEQ - HALF, HALF)]
        xout_v[pl.ds(r * SEQ, HALF)] = a
        xout_v[pl.ds(r * SEQ + (SEQ - HALF), HALF)] = b

    pltpu.sync_copy(xout_v, out_hbm.at[pl.ds(wid * IDX_PER_W, IDX_PER_W)])


@functools.partial(
    pl.kernel,
    out_type=jax.ShapeDtypeStruct((BATCH, EMBED_DIM), jnp.float32),
    mesh=_MESH,
    compiler_params=pltpu.CompilerParams(use_tc_tiling_on_sc=False),
    scratch_types=[
        pltpu.VMEM((IDX_PER_W,), jnp.int32),
        pltpu.VMEM((CHUNK_IDX, EMBED_DIM), jnp.float32),
        pltpu.VMEM((CHUNK_IDX, EMBED_DIM), jnp.float32),
        pltpu.VMEM((B_PER_W, EMBED_DIM), jnp.float32),
        pltpu.SemaphoreType.DMA,
        pltpu.SemaphoreType.DMA,
    ],
)
def _sc_embed_sum(table_hbm, idx_hbm, out_hbm, idx_v, rows0, rows1, out_v,
                  sem0, sem1):
    wid = _worker_id()
    base_b = wid * B_PER_W
    pltpu.sync_copy(idx_hbm.at[pl.ds(wid * IDX_PER_W, IDX_PER_W)], idx_v)

    rows = (rows0, rows1)
    sems = (sem0, sem1)

    def fire(c):
        buf, sem = rows[c % 2], sems[c % 2]
        cps = []
        for g in range(CHUNK_STREAMS):
            cps.append(pltpu.async_copy(
                table_hbm.at[idx_v.at[pl.ds(c * CHUNK_IDX + g * LANES, LANES)]],
                buf.at[pl.ds(g * LANES, LANES)],
                sem))
        return cps

    pending = fire(0)
    for c in range(NUM_CHUNKS):
        for cp in pending:
            cp.wait()
        if c + 1 < NUM_CHUNKS:
            pending = fire(c + 1)
        buf = rows[c % 2]

        @pl.loop(0, CHUNK_B)
        def _(b, _c=c, _buf=buf):
            r0 = b * SEQ
            acc0 = _buf[r0, pl.ds(0, HALF)]
            acc1 = _buf[r0, pl.ds(HALF, HALF)]
            for l in range(1, SEQ):
                acc0 = acc0 + _buf[r0 + l, pl.ds(0, HALF)]
                acc1 = acc1 + _buf[r0 + l, pl.ds(HALF, HALF)]
            ob = _c * CHUNK_B + b
            out_v[ob, pl.ds(0, HALF)] = acc0
            out_v[ob, pl.ds(HALF, HALF)] = acc1

    pltpu.sync_copy(out_v, out_hbm.at[pl.ds(base_b, B_PER_W)])


def kernel(x, action_emb):
    x_flat = _sc_flatten(x.astype(jnp.int32))
    return _sc_embed_sum(action_emb, x_flat)
